# Initial kernel scaffold; baseline (speedup 1.0000x reference)
import functools

import jax
import jax.numpy as jnp
from jax import lax
from jax.experimental import pallas as pl
from jax.experimental.pallas import tpu as pltpu
from jax.experimental.pallas import tpu_sc as plsc

N = 10000
D = 128
E = 320000
NC = 2
NS = 16
NW = NC * NS
EPW = E // NW
CHUNK = 80
NCHUNKS = EPW // CHUNK
RPT = N // NS
RCH = 125
NRCH = RPT // RCH

MMB = 2000


def _mm_body(x_ref, w_ref, ws_ref, b_ref, s1_ref, s2_ref):
    xb = x_ref[...]
    s1_ref[...] = jnp.dot(xb, w_ref[...], preferred_element_type=jnp.float32,
                          precision=lax.Precision.HIGHEST)
    s2_ref[...] = jnp.dot(xb, ws_ref[...], preferred_element_type=jnp.float32,
                          precision=lax.Precision.HIGHEST) + b_ref[...]


def _combine_body(agg_ref, s2_ref, o_ref):
    o_ref[...] = agg_ref[0] + agg_ref[1] + s2_ref[...]


def _sc_body(s1_hbm, src_hbm, dst_hbm, w_hbm, zero_hbm, out_hbm,
             agg_sh, rows_v, bounce_v, src_v, dst_v, w_v, sem):
    c = lax.axis_index("c")
    s = lax.axis_index("s")
    wid = c * NS + s

    pltpu.sync_copy(zero_hbm, bounce_v)
    for r in range(NRCH):
        pltpu.sync_copy(bounce_v, agg_sh.at[pl.ds(s * RPT + r * RCH, RCH)])
    plsc.subcore_barrier()

    def chunk_body(g, carry):
        base = pl.multiple_of(wid * EPW + g * CHUNK, CHUNK)
        pltpu.sync_copy(src_hbm.at[pl.ds(base, CHUNK)], src_v)
        pltpu.sync_copy(dst_hbm.at[pl.ds(base, CHUNK)], dst_v)
        pltpu.sync_copy(w_hbm.at[pl.ds(base, CHUNK)], w_v)
        pltpu.async_copy(s1_hbm.at[src_v], rows_v, sem).wait()

        def scale(i, carry2):
            wsp = plsc.load_gather(w_v, [jnp.full((16,), i, jnp.int32)])
            for k in range(8):
                rows_v[i, pl.ds(k * 16, 16)] = rows_v[i, pl.ds(k * 16, 16)] * wsp
            return carry2

        lax.fori_loop(0, CHUNK, scale, 0)
        pltpu.sync_copy(rows_v, agg_sh.at[dst_v], add=True)
        return carry

    lax.fori_loop(0, NCHUNKS, chunk_body, 0)
    plsc.subcore_barrier()

    for r in range(NRCH):
        row0 = s * RPT + r * RCH
        pltpu.sync_copy(agg_sh.at[pl.ds(row0, RCH)], bounce_v)
        pltpu.sync_copy(bounce_v, out_hbm.at[c, pl.ds(row0, RCH)])


_sc_spmm = functools.partial(
    pl.kernel,
    out_type=jax.ShapeDtypeStruct((NC, N, D), jnp.float32),
    mesh=plsc.VectorSubcoreMesh(core_axis_name="c", subcore_axis_name="s"),
    scratch_types=[
        pltpu.VMEM_SHARED((N, D), jnp.float32),
        pltpu.VMEM((CHUNK, D), jnp.float32),
        pltpu.VMEM((RCH, D), jnp.float32),
        pltpu.VMEM((CHUNK,), jnp.int32),
        pltpu.VMEM((CHUNK,), jnp.int32),
        pltpu.VMEM((CHUNK,), jnp.float32),
        pltpu.SemaphoreType.DMA,
    ],
)(_sc_body)


def kernel(x, edge_index, edge_weight, W, W_skip, b):
    src = edge_index[0].astype(jnp.int32)
    dst = edge_index[1].astype(jnp.int32)
    ew = edge_weight.astype(jnp.float32)
    b2 = b.reshape(1, D)

    support1, support2b = pl.pallas_call(
        _mm_body,
        grid=(N // MMB,),
        in_specs=[
            pl.BlockSpec((MMB, D), lambda i: (i, 0)),
            pl.BlockSpec((D, D), lambda i: (0, 0)),
            pl.BlockSpec((D, D), lambda i: (0, 0)),
            pl.BlockSpec((1, D), lambda i: (0, 0)),
        ],
        out_specs=[
            pl.BlockSpec((MMB, D), lambda i: (i, 0)),
            pl.BlockSpec((MMB, D), lambda i: (i, 0)),
        ],
        out_shape=[
            jax.ShapeDtypeStruct((N, D), jnp.float32),
            jax.ShapeDtypeStruct((N, D), jnp.float32),
        ],
    )(x, W, W_skip, b2)

    zero_block = jnp.zeros((RCH, D), jnp.float32)
    agg = _sc_spmm(support1, src, dst, ew, zero_block)

    out = pl.pallas_call(
        _combine_body,
        grid=(N // MMB,),
        in_specs=[
            pl.BlockSpec((NC, MMB, D), lambda i: (0, i, 0)),
            pl.BlockSpec((MMB, D), lambda i: (i, 0)),
        ],
        out_specs=pl.BlockSpec((MMB, D), lambda i: (i, 0)),
        out_shape=jax.ShapeDtypeStruct((N, D), jnp.float32),
    )(agg, support2b)
    return out


# speedup vs baseline: 4.2655x; 4.2655x over previous
import functools

import jax
import jax.numpy as jnp
from jax import lax
from jax.experimental import pallas as pl
from jax.experimental.pallas import tpu as pltpu
from jax.experimental.pallas import tpu_sc as plsc

N = 10000
D = 128
E = 320000
NC = 2
NS = 16
NW = NC * NS
EPW = E // NW
CHUNK = 80
NCHUNKS = EPW // CHUNK
RCH = 80
TOTRCH = N // RCH
RPASS = -(-TOTRCH // NS)

MMB = 2000


def _mm_body(x_ref, w_ref, ws_ref, b_ref, s1_ref, s2_ref):
    xb = x_ref[...]
    s1_ref[...] = jnp.dot(xb, w_ref[...], preferred_element_type=jnp.float32,
                          precision=lax.Precision.HIGHEST)
    s2_ref[...] = jnp.dot(xb, ws_ref[...], preferred_element_type=jnp.float32,
                          precision=lax.Precision.HIGHEST) + b_ref[...]


def _combine_body(agg_ref, s2_ref, o_ref):
    o_ref[...] = agg_ref[0] + agg_ref[1] + s2_ref[...]


def _sc_body(s1_hbm, src_hbm, dst_hbm, w_hbm, zero_hbm, out_hbm,
             agg_sh, rows_v, bounce_v, src_v, dst_v, w_v, sem):
    c = lax.axis_index("c")
    s = lax.axis_index("s")
    wid = c * NS + s

    pltpu.sync_copy(zero_hbm, bounce_v)
    for r in range(RPASS):
        cid = r * NS + s

        @pl.when(cid < TOTRCH)
        def _():
            row0 = pl.multiple_of(cid * RCH, RCH)
            pltpu.sync_copy(bounce_v, agg_sh.at[pl.ds(row0, RCH)])

    plsc.subcore_barrier()

    def chunk_body(g, carry):
        base = pl.multiple_of(wid * EPW + g * CHUNK, CHUNK)
        pltpu.sync_copy(src_hbm.at[pl.ds(base, CHUNK)], src_v)
        pltpu.sync_copy(dst_hbm.at[pl.ds(base, CHUNK)], dst_v)
        pltpu.sync_copy(w_hbm.at[pl.ds(base, CHUNK)], w_v)
        pltpu.async_copy(s1_hbm.at[src_v], rows_v, sem).wait()

        def scale(g2, carry2):
            w16 = w_v[pl.ds(g2 * 16, 16)]
            for l in range(16):
                w_s = w16[l]
                i = g2 * 16 + l
                for k in range(8):
                    rows_v[i, pl.ds(k * 16, 16)] = rows_v[i, pl.ds(k * 16, 16)] * w_s
            return carry2

        lax.fori_loop(0, CHUNK // 16, scale, 0)
        pltpu.sync_copy(rows_v, agg_sh.at[dst_v], add=True)
        return carry

    lax.fori_loop(0, NCHUNKS, chunk_body, 0)
    plsc.subcore_barrier()

    for r in range(RPASS):
        cid = r * NS + s

        @pl.when(cid < TOTRCH)
        def _():
            row0 = pl.multiple_of(cid * RCH, RCH)
            pltpu.sync_copy(agg_sh.at[pl.ds(row0, RCH)], bounce_v)
            pltpu.sync_copy(bounce_v, out_hbm.at[c, pl.ds(row0, RCH)])


_sc_spmm = functools.partial(
    pl.kernel,
    out_type=jax.ShapeDtypeStruct((NC, N, D), jnp.float32),
    mesh=plsc.VectorSubcoreMesh(core_axis_name="c", subcore_axis_name="s"),
    scratch_types=[
        pltpu.VMEM_SHARED((N, D), jnp.float32),
        pltpu.VMEM((CHUNK, D), jnp.float32),
        pltpu.VMEM((RCH, D), jnp.float32),
        pltpu.VMEM((CHUNK,), jnp.int32),
        pltpu.VMEM((CHUNK,), jnp.int32),
        pltpu.VMEM((CHUNK,), jnp.float32),
        pltpu.SemaphoreType.DMA,
    ],
)(_sc_body)


def kernel(x, edge_index, edge_weight, W, W_skip, b):
    src = edge_index[0].astype(jnp.int32)
    dst = edge_index[1].astype(jnp.int32)
    ew = edge_weight.astype(jnp.float32)
    b2 = b.reshape(1, D)

    support1, support2b = pl.pallas_call(
        _mm_body,
        grid=(N // MMB,),
        in_specs=[
            pl.BlockSpec((MMB, D), lambda i: (i, 0)),
            pl.BlockSpec((D, D), lambda i: (0, 0)),
            pl.BlockSpec((D, D), lambda i: (0, 0)),
            pl.BlockSpec((1, D), lambda i: (0, 0)),
        ],
        out_specs=[
            pl.BlockSpec((MMB, D), lambda i: (i, 0)),
            pl.BlockSpec((MMB, D), lambda i: (i, 0)),
        ],
        out_shape=[
            jax.ShapeDtypeStruct((N, D), jnp.float32),
            jax.ShapeDtypeStruct((N, D), jnp.float32),
        ],
    )(x, W, W_skip, b2)

    zero_block = jnp.zeros((RCH, D), jnp.float32)
    agg = _sc_spmm(support1, src, dst, ew, zero_block)

    out = pl.pallas_call(
        _combine_body,
        grid=(N // MMB,),
        in_specs=[
            pl.BlockSpec((NC, MMB, D), lambda i: (0, i, 0)),
            pl.BlockSpec((MMB, D), lambda i: (i, 0)),
        ],
        out_specs=pl.BlockSpec((MMB, D), lambda i: (i, 0)),
        out_shape=jax.ShapeDtypeStruct((N, D), jnp.float32),
    )(agg, support2b)
    return out


# speedup vs baseline: 10.2082x; 2.3932x over previous
import functools

import jax
import jax.numpy as jnp
from jax import lax
from jax.experimental import pallas as pl
from jax.experimental.pallas import tpu as pltpu
from jax.experimental.pallas import tpu_sc as plsc

N = 10000
D = 128
E = 320000
NC = 2
NS = 16
NW = NC * NS
EPW = E // NW
CHUNK = 80
NCHUNKS = EPW // CHUNK
NBUF = 2
RCH = 80
TOTRCH = N // RCH
RPASS = -(-TOTRCH // NS)

MMB = 2000


def _mm_body(x_ref, w_ref, ws_ref, b_ref, s1_ref, s2_ref):
    xb = x_ref[...]
    s1_ref[...] = jnp.dot(xb, w_ref[...], preferred_element_type=jnp.float32,
                          precision=lax.Precision.HIGHEST)
    s2_ref[...] = jnp.dot(xb, ws_ref[...], preferred_element_type=jnp.float32,
                          precision=lax.Precision.HIGHEST) + b_ref[...]


def _combine_body(agg_ref, s2_ref, o_ref):
    o_ref[...] = agg_ref[0] + agg_ref[1] + s2_ref[...]


def _sc_body(s1_hbm, src_hbm, dst_hbm, w_hbm, zero_hbm, out_hbm,
             agg_sh, r0, r1, src_v, dst_v, w_v, dst80, gsem, lsem):
    bounce_v = r1
    c = lax.axis_index("c")
    s = lax.axis_index("s")
    wid = c * NS + s
    rows = [r0, r1]

    ebase = pl.multiple_of(wid * EPW, EPW)
    cp_src = pltpu.async_copy(src_hbm.at[pl.ds(ebase, EPW)], src_v, lsem)
    cp_dst = pltpu.async_copy(dst_hbm.at[pl.ds(ebase, EPW)], dst_v, lsem)
    cp_w = pltpu.async_copy(w_hbm.at[pl.ds(ebase, EPW)], w_v, lsem)

    pltpu.sync_copy(zero_hbm, bounce_v)
    for r in range(RPASS):
        cid = r * NS + s

        @pl.when(cid < TOTRCH)
        def _():
            row0 = pl.multiple_of(cid * RCH, RCH)
            pltpu.sync_copy(bounce_v, agg_sh.at[pl.ds(row0, RCH)])

    cp_src.wait()
    cp_dst.wait()
    cp_w.wait()

    for b in range(NBUF):
        pltpu.async_copy(
            s1_hbm.at[src_v.at[pl.ds(b * CHUNK, CHUNK)]], rows[b],
            gsem.at[b])

    plsc.subcore_barrier()

    def do_chunk(g, b):
        pltpu.make_async_copy(
            s1_hbm.at[pl.ds(0, CHUNK)], rows[b], gsem.at[b]).wait()

        def scale(j, c2, _b=b):
            w16 = w_v[pl.ds(g * CHUNK + j * 16, 16)]
            rbuf = rows[_b]
            for l in range(16):
                w_s = w16[l]
                row = j * 16 + l
                for k in range(D // 16):
                    rbuf[row, pl.ds(k * 16, 16)] = (
                        rbuf[row, pl.ds(k * 16, 16)] * w_s)
            return c2

        lax.fori_loop(0, CHUNK // 16, scale, 0)

        for j in range(CHUNK // 16):
            dst80[pl.ds(j * 16, 16)] = dst_v[pl.ds(g * CHUNK + j * 16, 16)]

        pltpu.sync_copy(rows[b], agg_sh.at[dst80], add=True)

        @pl.when(g + NBUF < NCHUNKS)
        def _(_b=b):
            pltpu.async_copy(
                s1_hbm.at[src_v.at[pl.ds((g + NBUF) * CHUNK, CHUNK)]],
                rows[_b], gsem.at[_b])

    def outer(g0, carry):
        do_chunk(g0 * NBUF, 0)
        do_chunk(g0 * NBUF + 1, 1)
        return carry

    lax.fori_loop(0, NCHUNKS // NBUF, outer, 0)
    do_chunk(NCHUNKS - 1, 0)
    plsc.subcore_barrier()

    for r in range(RPASS):
        cid = r * NS + s

        @pl.when(cid < TOTRCH)
        def _():
            row0 = pl.multiple_of(cid * RCH, RCH)
            pltpu.sync_copy(agg_sh.at[pl.ds(row0, RCH)], bounce_v)
            pltpu.sync_copy(bounce_v, out_hbm.at[c, pl.ds(row0, RCH)])


_sc_spmm = functools.partial(
    pl.kernel,
    out_type=jax.ShapeDtypeStruct((NC, N, D), jnp.float32),
    mesh=plsc.VectorSubcoreMesh(core_axis_name="c", subcore_axis_name="s"),
    scratch_types=[
        pltpu.VMEM_SHARED((N, D), jnp.float32),
        pltpu.VMEM((CHUNK, D), jnp.float32),
        pltpu.VMEM((CHUNK, D), jnp.float32),
        pltpu.VMEM((EPW,), jnp.int32),
        pltpu.VMEM((EPW,), jnp.int32),
        pltpu.VMEM((EPW,), jnp.float32),
        pltpu.VMEM((CHUNK,), jnp.int32),
        pltpu.SemaphoreType.DMA((NBUF,)),
        pltpu.SemaphoreType.DMA,
    ],
)(_sc_body)


def kernel(x, edge_index, edge_weight, W, W_skip, b):
    src = edge_index[0].astype(jnp.int32)
    dst = edge_index[1].astype(jnp.int32)
    ew = edge_weight.astype(jnp.float32)
    b2 = b.reshape(1, D)

    support1, support2b = pl.pallas_call(
        _mm_body,
        grid=(N // MMB,),
        in_specs=[
            pl.BlockSpec((MMB, D), lambda i: (i, 0)),
            pl.BlockSpec((D, D), lambda i: (0, 0)),
            pl.BlockSpec((D, D), lambda i: (0, 0)),
            pl.BlockSpec((1, D), lambda i: (0, 0)),
        ],
        out_specs=[
            pl.BlockSpec((MMB, D), lambda i: (i, 0)),
            pl.BlockSpec((MMB, D), lambda i: (i, 0)),
        ],
        out_shape=[
            jax.ShapeDtypeStruct((N, D), jnp.float32),
            jax.ShapeDtypeStruct((N, D), jnp.float32),
        ],
    )(x, W, W_skip, b2)

    zero_block = jnp.zeros((RCH, D), jnp.float32)
    agg = _sc_spmm(support1, src, dst, ew, zero_block)

    out = pl.pallas_call(
        _combine_body,
        grid=(N // MMB,),
        in_specs=[
            pl.BlockSpec((NC, MMB, D), lambda i: (0, i, 0)),
            pl.BlockSpec((MMB, D), lambda i: (i, 0)),
        ],
        out_specs=pl.BlockSpec((MMB, D), lambda i: (i, 0)),
        out_shape=jax.ShapeDtypeStruct((N, D), jnp.float32),
    )(agg, support2b)
    return out


# speedup vs baseline: 11.3437x; 1.1112x over previous
import functools

import jax
import jax.numpy as jnp
from jax import lax
from jax.experimental import pallas as pl
from jax.experimental.pallas import tpu as pltpu
from jax.experimental.pallas import tpu_sc as plsc

N = 10000
D = 128
E = 320000
NC = 2
NS = 16
NW = NC * NS
EPW = E // NW
CHUNK = 80
NCHUNKS = EPW // CHUNK
NBUF = 3
RCH = 80
TOTRCH = N // RCH
RPASS = -(-TOTRCH // NS)

MMB = 2000


def _mm_body(x_ref, w_ref, ws_ref, b_ref, s1_ref, s2_ref):
    xb = x_ref[...]
    s1_ref[...] = jnp.dot(xb, w_ref[...], preferred_element_type=jnp.float32,
                          precision=lax.Precision.HIGHEST)
    s2_ref[...] = jnp.dot(xb, ws_ref[...], preferred_element_type=jnp.float32,
                          precision=lax.Precision.HIGHEST) + b_ref[...]


def _combine_body(agg_ref, s2_ref, o_ref):
    o_ref[...] = agg_ref[0] + agg_ref[1] + s2_ref[...]


def _sc_body(s1_hbm, src_hbm, dst_hbm, w_hbm, zero_hbm, out_hbm,
             agg_sh, r0, r1, r2, src_v, d0, d1, d2, w0, w1, w2,
             gsem, isem, ssem, lsem):
    bounce_v = r2
    c = lax.axis_index("c")
    s = lax.axis_index("s")
    wid = c * NS + s
    rows = [r0, r1, r2]
    dbuf = [d0, d1, d2]
    wbuf = [w0, w1, w2]
    ebase = pl.multiple_of(wid * EPW, EPW)

    def issue_idx(g, sl):
        off = pl.multiple_of(ebase + g * CHUNK, CHUNK)
        pltpu.async_copy(dst_hbm.at[pl.ds(off, CHUNK)], dbuf[sl], isem.at[sl])
        pltpu.async_copy(w_hbm.at[pl.ds(off, CHUNK)], wbuf[sl], isem.at[sl])

    def wait_idx(g, sl):
        off = pl.multiple_of(ebase + g * CHUNK, CHUNK)
        pltpu.make_async_copy(
            dst_hbm.at[pl.ds(off, CHUNK)], dbuf[sl], isem.at[sl]).wait()
        pltpu.make_async_copy(
            w_hbm.at[pl.ds(off, CHUNK)], wbuf[sl], isem.at[sl]).wait()

    def issue_gather(g, sl):
        pltpu.async_copy(
            s1_hbm.at[src_v.at[pl.ds(g * CHUNK, CHUNK)]], rows[sl],
            gsem.at[sl])

    def wait_gather(sl):
        pltpu.make_async_copy(
            s1_hbm.at[pl.ds(0, CHUNK)], rows[sl], gsem.at[sl]).wait()

    def issue_scatter(sl):
        pltpu.async_copy(rows[sl], agg_sh.at[dbuf[sl]], ssem.at[sl],
                         add=True)

    def wait_scatter(sl):
        pltpu.make_async_copy(rows[sl], agg_sh.at[dbuf[sl]],
                              ssem.at[sl]).wait()

    cp_src = pltpu.async_copy(src_hbm.at[pl.ds(ebase, EPW)], src_v, lsem)

    pltpu.sync_copy(zero_hbm, bounce_v)
    for r in range(RPASS):
        cid = r * NS + s

        @pl.when(cid < TOTRCH)
        def _():
            row0 = pl.multiple_of(cid * RCH, RCH)
            pltpu.sync_copy(bounce_v, agg_sh.at[pl.ds(row0, RCH)])

    cp_src.wait()

    for b in range(NBUF - 1):
        issue_idx(b, b)
        issue_gather(b, b)

    plsc.subcore_barrier()

    def do_chunk(g, b):
        g = jnp.int32(g)
        bs = (b + NBUF - 1) % NBUF

        wait_gather(b)
        wait_idx(g, b)

        def scale(j, c2, _b=b):
            w16 = wbuf[_b][pl.ds(j * 16, 16)]
            rbuf = rows[_b]
            for l in range(16):
                w_s = w16[l]
                row = j * 16 + l
                for k in range(D // 16):
                    rbuf[row, pl.ds(k * 16, 16)] = (
                        rbuf[row, pl.ds(k * 16, 16)] * w_s)
            return c2

        lax.fori_loop(0, CHUNK // 16, scale, 0)

        @pl.when(g >= 1)
        def _():
            wait_scatter(bs)

        @pl.when(g + 2 < NCHUNKS)
        def _():
            issue_idx(g + 2, bs)
            issue_gather(g + 2, bs)

        issue_scatter(b)

    def outer(g0, carry):
        for b in range(NBUF):
            do_chunk(g0 * NBUF + b, b)
        return carry

    n_main = (NCHUNKS // NBUF) * NBUF
    lax.fori_loop(0, NCHUNKS // NBUF, outer, 0)
    for t in range(NCHUNKS - n_main):
        do_chunk(n_main + t, t)

    wait_scatter((NCHUNKS - 1) % NBUF)
    plsc.subcore_barrier()

    for r in range(RPASS):
        cid = r * NS + s

        @pl.when(cid < TOTRCH)
        def _():
            row0 = pl.multiple_of(cid * RCH, RCH)
            pltpu.sync_copy(agg_sh.at[pl.ds(row0, RCH)], bounce_v)
            pltpu.sync_copy(bounce_v, out_hbm.at[c, pl.ds(row0, RCH)])


_sc_spmm = functools.partial(
    pl.kernel,
    out_type=jax.ShapeDtypeStruct((NC, N, D), jnp.float32),
    mesh=plsc.VectorSubcoreMesh(core_axis_name="c", subcore_axis_name="s"),
    scratch_types=[
        pltpu.VMEM_SHARED((N, D), jnp.float32),
        pltpu.VMEM((CHUNK, D), jnp.float32),
        pltpu.VMEM((CHUNK, D), jnp.float32),
        pltpu.VMEM((CHUNK, D), jnp.float32),
        pltpu.VMEM((EPW,), jnp.int32),
        pltpu.VMEM((CHUNK,), jnp.int32),
        pltpu.VMEM((CHUNK,), jnp.int32),
        pltpu.VMEM((CHUNK,), jnp.int32),
        pltpu.VMEM((CHUNK,), jnp.float32),
        pltpu.VMEM((CHUNK,), jnp.float32),
        pltpu.VMEM((CHUNK,), jnp.float32),
        pltpu.SemaphoreType.DMA((NBUF,)),
        pltpu.SemaphoreType.DMA((NBUF,)),
        pltpu.SemaphoreType.DMA((NBUF,)),
        pltpu.SemaphoreType.DMA,
    ],
)(_sc_body)


def kernel(x, edge_index, edge_weight, W, W_skip, b):
    src = edge_index[0].astype(jnp.int32)
    dst = edge_index[1].astype(jnp.int32)
    ew = edge_weight.astype(jnp.float32)
    b2 = b.reshape(1, D)

    support1, support2b = pl.pallas_call(
        _mm_body,
        grid=(N // MMB,),
        in_specs=[
            pl.BlockSpec((MMB, D), lambda i: (i, 0)),
            pl.BlockSpec((D, D), lambda i: (0, 0)),
            pl.BlockSpec((D, D), lambda i: (0, 0)),
            pl.BlockSpec((1, D), lambda i: (0, 0)),
        ],
        out_specs=[
            pl.BlockSpec((MMB, D), lambda i: (i, 0)),
            pl.BlockSpec((MMB, D), lambda i: (i, 0)),
        ],
        out_shape=[
            jax.ShapeDtypeStruct((N, D), jnp.float32),
            jax.ShapeDtypeStruct((N, D), jnp.float32),
        ],
    )(x, W, W_skip, b2)

    zero_block = jnp.zeros((RCH, D), jnp.float32)
    agg = _sc_spmm(support1, src, dst, ew, zero_block)

    out = pl.pallas_call(
        _combine_body,
        grid=(N // MMB,),
        in_specs=[
            pl.BlockSpec((NC, MMB, D), lambda i: (0, i, 0)),
            pl.BlockSpec((MMB, D), lambda i: (i, 0)),
        ],
        out_specs=pl.BlockSpec((MMB, D), lambda i: (i, 0)),
        out_shape=jax.ShapeDtypeStruct((N, D), jnp.float32),
    )(agg, support2b)
    return out


# speedup vs baseline: 11.4345x; 1.0080x over previous
import functools

import jax
import jax.numpy as jnp
from jax import lax
from jax.experimental import pallas as pl
from jax.experimental.pallas import tpu as pltpu
from jax.experimental.pallas import tpu_sc as plsc

N = 10000
D = 128
E = 320000
NC = 2
NS = 16
NW = NC * NS
EPW = E // NW
CHUNK = 80
NCHUNKS = EPW // CHUNK
NBUF = 3
RCH = 80
TOTRCH = N // RCH
RPASS = -(-TOTRCH // NS)

MMB = 2000


def _mm_body(x_ref, w_ref, ws_ref, b_ref, s1_ref, s2_ref):
    xb = x_ref[...]
    s1_ref[...] = jnp.dot(xb, w_ref[...], preferred_element_type=jnp.float32,
                          precision=lax.Precision.HIGHEST)
    s2_ref[...] = jnp.dot(xb, ws_ref[...], preferred_element_type=jnp.float32,
                          precision=lax.Precision.HIGHEST) + b_ref[...]


def _combine_body(agg_ref, s2_ref, o_ref):
    o_ref[...] = agg_ref[0] + agg_ref[1] + s2_ref[...]


def _sc_body(s1_hbm, src_hbm, dst_hbm, w_hbm, zero_hbm, out_hbm,
             agg_sh, r0, r1, r2, src_v, d0, d1, d2, w0, w1, w2,
             gsem, isem, ssem, lsem):
    bounce_v = r2
    c = lax.axis_index("c")
    s = lax.axis_index("s")
    wid = c * NS + s
    rows = [r0, r1, r2]
    dbuf = [d0, d1, d2]
    wbuf = [w0, w1, w2]
    ebase = pl.multiple_of(wid * EPW, EPW)

    def issue_idx(g, sl):
        off = pl.multiple_of(ebase + g * CHUNK, CHUNK)
        pltpu.async_copy(dst_hbm.at[pl.ds(off, CHUNK)], dbuf[sl], isem.at[sl])
        pltpu.async_copy(w_hbm.at[pl.ds(off, CHUNK)], wbuf[sl], isem.at[sl])

    def wait_idx(g, sl):
        off = pl.multiple_of(ebase + g * CHUNK, CHUNK)
        pltpu.make_async_copy(
            dst_hbm.at[pl.ds(off, CHUNK)], dbuf[sl], isem.at[sl]).wait()
        pltpu.make_async_copy(
            w_hbm.at[pl.ds(off, CHUNK)], wbuf[sl], isem.at[sl]).wait()

    def issue_gather(g, sl):
        pltpu.async_copy(
            s1_hbm.at[src_v.at[pl.ds(g * CHUNK, CHUNK)]], rows[sl],
            gsem.at[sl])

    def wait_gather(sl):
        pltpu.make_async_copy(
            s1_hbm.at[pl.ds(0, CHUNK)], rows[sl], gsem.at[sl]).wait()

    def issue_scatter(sl):
        pltpu.async_copy(rows[sl], agg_sh.at[dbuf[sl]], ssem.at[sl],
                         add=True)

    def wait_scatter(sl):
        pltpu.make_async_copy(rows[sl], agg_sh.at[dbuf[sl]],
                              ssem.at[sl]).wait()

    cp_src = pltpu.async_copy(src_hbm.at[pl.ds(ebase, EPW)], src_v, lsem)

    pltpu.sync_copy(zero_hbm, bounce_v)
    for r in range(RPASS):
        cid = r * NS + s

        @pl.when(cid < TOTRCH)
        def _():
            row0 = pl.multiple_of(cid * RCH, RCH)
            pltpu.sync_copy(bounce_v, agg_sh.at[pl.ds(row0, RCH)])

    cp_src.wait()

    for b in range(NBUF - 1):
        issue_idx(b, b)
        issue_gather(b, b)

    plsc.subcore_barrier()

    def do_chunk(g, b):
        g = jnp.int32(g)
        bs = (b + NBUF - 1) % NBUF

        wait_gather(b)
        wait_idx(g, b)

        lane = [jnp.full((16,), l, jnp.int32) for l in range(16)]

        def scale(j, c2, _b=b):
            w16 = wbuf[_b][pl.ds(j * 16, 16)]
            rbuf = rows[_b]
            for l in range(16):
                wsp = lax.gather(
                    w16, lane[l][:, None],
                    lax.GatherDimensionNumbers(
                        offset_dims=(), collapsed_slice_dims=(0,),
                        start_index_map=(0,)),
                    (1,), mode=lax.GatherScatterMode.PROMISE_IN_BOUNDS)
                row = j * 16 + l
                for k in range(D // 16):
                    rbuf[row, pl.ds(k * 16, 16)] = (
                        rbuf[row, pl.ds(k * 16, 16)] * wsp)
            return c2

        lax.fori_loop(0, CHUNK // 16, scale, 0)

        @pl.when(g >= 1)
        def _():
            wait_scatter(bs)

        @pl.when(g + 2 < NCHUNKS)
        def _():
            issue_idx(g + 2, bs)
            issue_gather(g + 2, bs)

        issue_scatter(b)

    def outer(g0, carry):
        for b in range(NBUF):
            do_chunk(g0 * NBUF + b, b)
        return carry

    n_main = (NCHUNKS // NBUF) * NBUF
    lax.fori_loop(0, NCHUNKS // NBUF, outer, 0)
    for t in range(NCHUNKS - n_main):
        do_chunk(n_main + t, t)

    wait_scatter((NCHUNKS - 1) % NBUF)
    plsc.subcore_barrier()

    for r in range(RPASS):
        cid = r * NS + s

        @pl.when(cid < TOTRCH)
        def _():
            row0 = pl.multiple_of(cid * RCH, RCH)
            pltpu.sync_copy(agg_sh.at[pl.ds(row0, RCH)], bounce_v)
            pltpu.sync_copy(bounce_v, out_hbm.at[c, pl.ds(row0, RCH)])


_sc_spmm = functools.partial(
    pl.kernel,
    out_type=jax.ShapeDtypeStruct((NC, N, D), jnp.float32),
    mesh=plsc.VectorSubcoreMesh(core_axis_name="c", subcore_axis_name="s"),
    scratch_types=[
        pltpu.VMEM_SHARED((N, D), jnp.float32),
        pltpu.VMEM((CHUNK, D), jnp.float32),
        pltpu.VMEM((CHUNK, D), jnp.float32),
        pltpu.VMEM((CHUNK, D), jnp.float32),
        pltpu.VMEM((EPW,), jnp.int32),
        pltpu.VMEM((CHUNK,), jnp.int32),
        pltpu.VMEM((CHUNK,), jnp.int32),
        pltpu.VMEM((CHUNK,), jnp.int32),
        pltpu.VMEM((CHUNK,), jnp.float32),
        pltpu.VMEM((CHUNK,), jnp.float32),
        pltpu.VMEM((CHUNK,), jnp.float32),
        pltpu.SemaphoreType.DMA((NBUF,)),
        pltpu.SemaphoreType.DMA((NBUF,)),
        pltpu.SemaphoreType.DMA((NBUF,)),
        pltpu.SemaphoreType.DMA,
    ],
)(_sc_body)


def kernel(x, edge_index, edge_weight, W, W_skip, b):
    src = edge_index[0].astype(jnp.int32)
    dst = edge_index[1].astype(jnp.int32)
    ew = edge_weight.astype(jnp.float32)
    b2 = b.reshape(1, D)

    support1, support2b = pl.pallas_call(
        _mm_body,
        grid=(N // MMB,),
        in_specs=[
            pl.BlockSpec((MMB, D), lambda i: (i, 0)),
            pl.BlockSpec((D, D), lambda i: (0, 0)),
            pl.BlockSpec((D, D), lambda i: (0, 0)),
            pl.BlockSpec((1, D), lambda i: (0, 0)),
        ],
        out_specs=[
            pl.BlockSpec((MMB, D), lambda i: (i, 0)),
            pl.BlockSpec((MMB, D), lambda i: (i, 0)),
        ],
        out_shape=[
            jax.ShapeDtypeStruct((N, D), jnp.float32),
            jax.ShapeDtypeStruct((N, D), jnp.float32),
        ],
    )(x, W, W_skip, b2)

    zero_block = jnp.zeros((RCH, D), jnp.float32)
    agg = _sc_spmm(support1, src, dst, ew, zero_block)

    out = pl.pallas_call(
        _combine_body,
        grid=(N // MMB,),
        in_specs=[
            pl.BlockSpec((NC, MMB, D), lambda i: (0, i, 0)),
            pl.BlockSpec((MMB, D), lambda i: (i, 0)),
        ],
        out_specs=pl.BlockSpec((MMB, D), lambda i: (i, 0)),
        out_shape=jax.ShapeDtypeStruct((N, D), jnp.float32),
    )(agg, support2b)
    return out


# speedup vs baseline: 11.8115x; 1.0330x over previous
import functools

import jax
import jax.numpy as jnp
from jax import lax
from jax.experimental import pallas as pl
from jax.experimental.pallas import tpu as pltpu
from jax.experimental.pallas import tpu_sc as plsc

N = 10000
D = 128
E = 320000
NC = 2
NS = 16
NW = NC * NS
EPW = E // NW
CHUNK = 40
NCHUNKS = EPW // CHUNK
NBUF = 6
RCH = 40
TOTRCH = N // RCH
RPASS = -(-TOTRCH // NS)

MMB = 2000


def _mm_body(x_ref, w_ref, ws_ref, b_ref, s1_ref, s2_ref):
    xb = x_ref[...]
    s1_ref[...] = jnp.dot(xb, w_ref[...], preferred_element_type=jnp.float32,
                          precision=lax.Precision.HIGHEST)
    s2_ref[...] = jnp.dot(xb, ws_ref[...], preferred_element_type=jnp.float32,
                          precision=lax.Precision.HIGHEST) + b_ref[...]


def _combine_body(agg_ref, s2_ref, o_ref):
    o_ref[...] = agg_ref[0] + agg_ref[1] + s2_ref[...]


def _sc_body(s1_hbm, src_hbm, dst_hbm, w_hbm, zero_hbm, out_hbm,
             agg_sh, r0, r1, r2, r3, r4, r5, src_v, d0, d1, d2, d3, d4, d5,
             w0, w1, w2, w3, w4, w5, gsem, isem, ssem, lsem):
    bounce_v = r5
    c = lax.axis_index("c")
    s = lax.axis_index("s")
    wid = c * NS + s
    rows = [r0, r1, r2, r3, r4, r5]
    dbuf = [d0, d1, d2, d3, d4, d5]
    wbuf = [w0, w1, w2, w3, w4, w5]
    ebase = pl.multiple_of(wid * EPW, EPW)

    def issue_idx(g, sl):
        off = pl.multiple_of(ebase + g * CHUNK, CHUNK)
        pltpu.async_copy(dst_hbm.at[pl.ds(off, CHUNK)], dbuf[sl], isem.at[sl])
        pltpu.async_copy(w_hbm.at[pl.ds(off, CHUNK)], wbuf[sl], isem.at[sl])

    def wait_idx(g, sl):
        off = pl.multiple_of(ebase + g * CHUNK, CHUNK)
        pltpu.make_async_copy(
            dst_hbm.at[pl.ds(off, CHUNK)], dbuf[sl], isem.at[sl]).wait()
        pltpu.make_async_copy(
            w_hbm.at[pl.ds(off, CHUNK)], wbuf[sl], isem.at[sl]).wait()

    def issue_gather(g, sl):
        pltpu.async_copy(
            s1_hbm.at[src_v.at[pl.ds(g * CHUNK, CHUNK)]], rows[sl],
            gsem.at[sl])

    def wait_gather(sl):
        pltpu.make_async_copy(
            s1_hbm.at[pl.ds(0, CHUNK)], rows[sl], gsem.at[sl]).wait()

    def issue_scatter(sl):
        pltpu.async_copy(rows[sl], agg_sh.at[dbuf[sl]], ssem.at[sl],
                         add=True)

    def wait_scatter(sl):
        pltpu.make_async_copy(rows[sl], agg_sh.at[dbuf[sl]],
                              ssem.at[sl]).wait()

    cp_src = pltpu.async_copy(src_hbm.at[pl.ds(ebase, EPW)], src_v, lsem)

    pltpu.sync_copy(zero_hbm, bounce_v)
    for r in range(RPASS):
        cid = r * NS + s

        @pl.when(cid < TOTRCH)
        def _():
            row0 = pl.multiple_of(cid * RCH, RCH)
            pltpu.sync_copy(bounce_v, agg_sh.at[pl.ds(row0, RCH)])

    cp_src.wait()

    for b in range(NBUF - 1):
        issue_idx(b, b)
        issue_gather(b, b)

    plsc.subcore_barrier()

    lane = [jnp.full((16,), l, jnp.int32) for l in range(16)]

    def do_chunk(g, b):
        g = jnp.int32(g)
        bs = (b + NBUF - 1) % NBUF

        wait_gather(b)
        wait_idx(g, b)

        def scale(j, c2, _b=b):
            w16 = wbuf[_b][pl.ds(j * 16, 16)]
            rbuf = rows[_b]
            for l in range(16):
                wsp = lax.gather(
                    w16, lane[l][:, None],
                    lax.GatherDimensionNumbers(
                        offset_dims=(), collapsed_slice_dims=(0,),
                        start_index_map=(0,)),
                    (1,), mode=lax.GatherScatterMode.PROMISE_IN_BOUNDS)
                row = j * 16 + l
                for k in range(D // 16):
                    rbuf[row, pl.ds(k * 16, 16)] = (
                        rbuf[row, pl.ds(k * 16, 16)] * wsp)
            return c2

        lax.fori_loop(0, CHUNK // 16, scale, 0)

        w16t = wbuf[b][pl.ds(CHUNK - 16, 16)]
        for l in range(8, 16):
            wsp_t = lax.gather(
                w16t, lane[l][:, None],
                lax.GatherDimensionNumbers(
                    offset_dims=(), collapsed_slice_dims=(0,),
                    start_index_map=(0,)),
                (1,), mode=lax.GatherScatterMode.PROMISE_IN_BOUNDS)
            row_t = CHUNK - 16 + l
            for k in range(D // 16):
                rows[b][row_t, pl.ds(k * 16, 16)] = (
                    rows[b][row_t, pl.ds(k * 16, 16)] * wsp_t)

        @pl.when(g >= 1)
        def _():
            wait_scatter(bs)

        @pl.when(g + NBUF - 1 < NCHUNKS)
        def _():
            issue_idx(g + NBUF - 1, bs)
            issue_gather(g + NBUF - 1, bs)

        issue_scatter(b)

    def outer(g0, carry):
        for b in range(NBUF):
            do_chunk(g0 * NBUF + b, b)
        return carry

    n_main = (NCHUNKS // NBUF) * NBUF
    lax.fori_loop(0, NCHUNKS // NBUF, outer, 0)
    for t in range(NCHUNKS - n_main):
        do_chunk(n_main + t, t)

    wait_scatter((NCHUNKS - 1) % NBUF)
    plsc.subcore_barrier()

    for r in range(RPASS):
        cid = r * NS + s

        @pl.when(cid < TOTRCH)
        def _():
            row0 = pl.multiple_of(cid * RCH, RCH)
            pltpu.sync_copy(agg_sh.at[pl.ds(row0, RCH)], bounce_v)
            pltpu.sync_copy(bounce_v, out_hbm.at[c, pl.ds(row0, RCH)])


_sc_spmm = functools.partial(
    pl.kernel,
    out_type=jax.ShapeDtypeStruct((NC, N, D), jnp.float32),
    mesh=plsc.VectorSubcoreMesh(core_axis_name="c", subcore_axis_name="s"),
    scratch_types=[
        pltpu.VMEM_SHARED((N, D), jnp.float32),
        pltpu.VMEM((CHUNK, D), jnp.float32),
        pltpu.VMEM((CHUNK, D), jnp.float32),
        pltpu.VMEM((CHUNK, D), jnp.float32),
        pltpu.VMEM((CHUNK, D), jnp.float32),
        pltpu.VMEM((CHUNK, D), jnp.float32),
        pltpu.VMEM((CHUNK, D), jnp.float32),
        pltpu.VMEM((EPW,), jnp.int32),
        pltpu.VMEM((CHUNK,), jnp.int32),
        pltpu.VMEM((CHUNK,), jnp.int32),
        pltpu.VMEM((CHUNK,), jnp.int32),
        pltpu.VMEM((CHUNK,), jnp.int32),
        pltpu.VMEM((CHUNK,), jnp.int32),
        pltpu.VMEM((CHUNK,), jnp.int32),
        pltpu.VMEM((CHUNK,), jnp.float32),
        pltpu.VMEM((CHUNK,), jnp.float32),
        pltpu.VMEM((CHUNK,), jnp.float32),
        pltpu.VMEM((CHUNK,), jnp.float32),
        pltpu.VMEM((CHUNK,), jnp.float32),
        pltpu.VMEM((CHUNK,), jnp.float32),
        pltpu.SemaphoreType.DMA((NBUF,)),
        pltpu.SemaphoreType.DMA((NBUF,)),
        pltpu.SemaphoreType.DMA((NBUF,)),
        pltpu.SemaphoreType.DMA,
    ],
)(_sc_body)


def kernel(x, edge_index, edge_weight, W, W_skip, b):
    src = edge_index[0].astype(jnp.int32)
    dst = edge_index[1].astype(jnp.int32)
    ew = edge_weight.astype(jnp.float32)
    b2 = b.reshape(1, D)

    support1, support2b = pl.pallas_call(
        _mm_body,
        grid=(N // MMB,),
        in_specs=[
            pl.BlockSpec((MMB, D), lambda i: (i, 0)),
            pl.BlockSpec((D, D), lambda i: (0, 0)),
            pl.BlockSpec((D, D), lambda i: (0, 0)),
            pl.BlockSpec((1, D), lambda i: (0, 0)),
        ],
        out_specs=[
            pl.BlockSpec((MMB, D), lambda i: (i, 0)),
            pl.BlockSpec((MMB, D), lambda i: (i, 0)),
        ],
        out_shape=[
            jax.ShapeDtypeStruct((N, D), jnp.float32),
            jax.ShapeDtypeStruct((N, D), jnp.float32),
        ],
    )(x, W, W_skip, b2)

    zero_block = jnp.zeros((RCH, D), jnp.float32)
    agg = _sc_spmm(support1, src, dst, ew, zero_block)

    out = pl.pallas_call(
        _combine_body,
        grid=(N // MMB,),
        in_specs=[
            pl.BlockSpec((NC, MMB, D), lambda i: (0, i, 0)),
            pl.BlockSpec((MMB, D), lambda i: (i, 0)),
        ],
        out_specs=pl.BlockSpec((MMB, D), lambda i: (i, 0)),
        out_shape=jax.ShapeDtypeStruct((N, D), jnp.float32),
    )(agg, support2b)
    return out


# speedup vs baseline: 12.0127x; 1.0170x over previous
import functools

import jax
import jax.numpy as jnp
from jax import lax
from jax.experimental import pallas as pl
from jax.experimental.pallas import tpu as pltpu
from jax.experimental.pallas import tpu_sc as plsc

N = 10000
D = 128
E = 320000
NC = 2
NS = 16
NW = NC * NS
EPW = E // NW
CHUNK = 40
NCHUNKS = EPW // CHUNK
NBUF = 6
RCH = 40
TOTRCH = N // RCH
RPASS = -(-TOTRCH // NS)

MMB = 2000


def _mm_body(x_ref, w_ref, ws_ref, b_ref, s1_ref, s2_ref):
    xb = x_ref[...]
    s1_ref[...] = jnp.dot(xb, w_ref[...], preferred_element_type=jnp.float32,
                          precision=lax.Precision.HIGHEST)
    s2_ref[...] = jnp.dot(xb, ws_ref[...], preferred_element_type=jnp.float32,
                          precision=lax.Precision.HIGHEST) + b_ref[...]


def _combine_body(agg_ref, s2_ref, o_ref):
    o_ref[...] = agg_ref[0] + agg_ref[1] + s2_ref[...]


def _sc_body(s1_hbm, src_hbm, dst_hbm, w_hbm, zero_hbm, out_hbm,
             agg_sh, r0, r1, r2, r3, r4, r5, src_v, d0, d1, d2, d3, d4, d5,
             w0, w1, w2, w3, w4, w5, gsem, isem, ssem, lsem):
    bounce_v = r5
    c = lax.axis_index("c")
    s = lax.axis_index("s")
    wid = c * NS + s
    rows = [r0, r1, r2, r3, r4, r5]
    dbuf = [d0, d1, d2, d3, d4, d5]
    wbuf = [w0, w1, w2, w3, w4, w5]
    ebase = pl.multiple_of(wid * EPW, EPW)

    def issue_idx(g, sl):
        off = pl.multiple_of(ebase + g * CHUNK, CHUNK)
        pltpu.async_copy(dst_hbm.at[pl.ds(off, CHUNK)], dbuf[sl], isem.at[sl])
        pltpu.async_copy(w_hbm.at[pl.ds(off, CHUNK)], wbuf[sl], isem.at[sl])

    def wait_idx(g, sl):
        off = pl.multiple_of(ebase + g * CHUNK, CHUNK)
        pltpu.make_async_copy(
            dst_hbm.at[pl.ds(off, CHUNK)], dbuf[sl], isem.at[sl]).wait()
        pltpu.make_async_copy(
            w_hbm.at[pl.ds(off, CHUNK)], wbuf[sl], isem.at[sl]).wait()

    def issue_gather(g, sl):
        pltpu.async_copy(
            s1_hbm.at[src_v.at[pl.ds(g * CHUNK, CHUNK)]], rows[sl],
            gsem.at[sl])

    def wait_gather(sl):
        pltpu.make_async_copy(
            s1_hbm.at[pl.ds(0, CHUNK)], rows[sl], gsem.at[sl]).wait()

    def issue_scatter(sl):
        pltpu.async_copy(rows[sl], agg_sh.at[dbuf[sl]], ssem.at[sl],
                         add=True)

    def wait_scatter(sl):
        pltpu.make_async_copy(rows[sl], agg_sh.at[dbuf[sl]],
                              ssem.at[sl]).wait()

    cp_src = pltpu.async_copy(src_hbm.at[pl.ds(ebase, EPW)], src_v, lsem)

    pltpu.sync_copy(zero_hbm, bounce_v)
    for r in range(RPASS):
        cid = r * NS + s

        @pl.when(cid < TOTRCH)
        def _():
            row0 = pl.multiple_of(cid * RCH, RCH)
            pltpu.sync_copy(bounce_v, agg_sh.at[pl.ds(row0, RCH)])

    cp_src.wait()

    for b in range(NBUF - 1):
        issue_idx(b, b)
        issue_gather(b, b)

    plsc.subcore_barrier()

    lane = [jnp.full((16,), l, jnp.int32) for l in range(16)]

    def do_chunk(g, b):
        g = jnp.int32(g)
        bs = (b + NBUF - 1) % NBUF

        wait_gather(b)
        wait_idx(g, b)

        def scale(j, c2, _b=b):
            w16 = wbuf[_b][pl.ds(j * 16, 16)]
            rbuf = rows[_b]
            for l in range(16):
                wsp = lax.gather(
                    w16, lane[l][:, None],
                    lax.GatherDimensionNumbers(
                        offset_dims=(), collapsed_slice_dims=(0,),
                        start_index_map=(0,)),
                    (1,), mode=lax.GatherScatterMode.PROMISE_IN_BOUNDS)
                row = j * 16 + l
                for k in range(D // 16):
                    rbuf[row, pl.ds(k * 16, 16)] = (
                        rbuf[row, pl.ds(k * 16, 16)] * wsp)
            return c2

        lax.fori_loop(0, CHUNK // 16, scale, 0)

        w16t = wbuf[b][pl.ds(CHUNK - 16, 16)]
        for l in range(8, 16):
            wsp_t = lax.gather(
                w16t, lane[l][:, None],
                lax.GatherDimensionNumbers(
                    offset_dims=(), collapsed_slice_dims=(0,),
                    start_index_map=(0,)),
                (1,), mode=lax.GatherScatterMode.PROMISE_IN_BOUNDS)
            row_t = CHUNK - 16 + l
            for k in range(D // 16):
                rows[b][row_t, pl.ds(k * 16, 16)] = (
                    rows[b][row_t, pl.ds(k * 16, 16)] * wsp_t)

        @pl.when(g >= 1)
        def _():
            wait_scatter(bs)

        @pl.when(g + NBUF - 1 < NCHUNKS)
        def _():
            issue_idx(g + NBUF - 1, bs)
            issue_gather(g + NBUF - 1, bs)

        issue_scatter(b)

    def outer(g0, carry):
        for b in range(NBUF):
            do_chunk(g0 * NBUF + b, b)
        return carry

    n_main = (NCHUNKS // NBUF) * NBUF
    lax.fori_loop(0, NCHUNKS // NBUF, outer, 0)
    for t in range(NCHUNKS - n_main):
        do_chunk(n_main + t, t)

    wait_scatter((NCHUNKS - 1) % NBUF)
    plsc.subcore_barrier()

    for r in range(RPASS):
        sl = r % NBUF
        cid = r * NS + s

        @pl.when(cid < TOTRCH)
        def _(sl=sl, cid=cid):
            row0 = pl.multiple_of(cid * RCH, RCH)
            if r >= NBUF:
                pltpu.make_async_copy(
                    rows[sl], out_hbm.at[c, pl.ds(0, RCH)],
                    ssem.at[sl]).wait()
            pltpu.sync_copy(agg_sh.at[pl.ds(row0, RCH)], rows[sl])
            pltpu.async_copy(rows[sl], out_hbm.at[c, pl.ds(row0, RCH)],
                             ssem.at[sl])

    for r in range(max(0, RPASS - NBUF), RPASS):
        sl = r % NBUF
        cid = r * NS + s

        @pl.when(cid < TOTRCH)
        def _(sl=sl):
            pltpu.make_async_copy(
                rows[sl], out_hbm.at[c, pl.ds(0, RCH)], ssem.at[sl]).wait()


_sc_spmm = functools.partial(
    pl.kernel,
    out_type=jax.ShapeDtypeStruct((NC, N, D), jnp.float32),
    mesh=plsc.VectorSubcoreMesh(core_axis_name="c", subcore_axis_name="s"),
    scratch_types=[
        pltpu.VMEM_SHARED((N, D), jnp.float32),
        pltpu.VMEM((CHUNK, D), jnp.float32),
        pltpu.VMEM((CHUNK, D), jnp.float32),
        pltpu.VMEM((CHUNK, D), jnp.float32),
        pltpu.VMEM((CHUNK, D), jnp.float32),
        pltpu.VMEM((CHUNK, D), jnp.float32),
        pltpu.VMEM((CHUNK, D), jnp.float32),
        pltpu.VMEM((EPW,), jnp.int32),
        pltpu.VMEM((CHUNK,), jnp.int32),
        pltpu.VMEM((CHUNK,), jnp.int32),
        pltpu.VMEM((CHUNK,), jnp.int32),
        pltpu.VMEM((CHUNK,), jnp.int32),
        pltpu.VMEM((CHUNK,), jnp.int32),
        pltpu.VMEM((CHUNK,), jnp.int32),
        pltpu.VMEM((CHUNK,), jnp.float32),
        pltpu.VMEM((CHUNK,), jnp.float32),
        pltpu.VMEM((CHUNK,), jnp.float32),
        pltpu.VMEM((CHUNK,), jnp.float32),
        pltpu.VMEM((CHUNK,), jnp.float32),
        pltpu.VMEM((CHUNK,), jnp.float32),
        pltpu.SemaphoreType.DMA((NBUF,)),
        pltpu.SemaphoreType.DMA((NBUF,)),
        pltpu.SemaphoreType.DMA((NBUF,)),
        pltpu.SemaphoreType.DMA,
    ],
)(_sc_body)


def kernel(x, edge_index, edge_weight, W, W_skip, b):
    src = edge_index[0].astype(jnp.int32)
    dst = edge_index[1].astype(jnp.int32)
    ew = edge_weight.astype(jnp.float32)
    b2 = b.reshape(1, D)

    support1, support2b = pl.pallas_call(
        _mm_body,
        grid=(N // MMB,),
        in_specs=[
            pl.BlockSpec((MMB, D), lambda i: (i, 0)),
            pl.BlockSpec((D, D), lambda i: (0, 0)),
            pl.BlockSpec((D, D), lambda i: (0, 0)),
            pl.BlockSpec((1, D), lambda i: (0, 0)),
        ],
        out_specs=[
            pl.BlockSpec((MMB, D), lambda i: (i, 0)),
            pl.BlockSpec((MMB, D), lambda i: (i, 0)),
        ],
        out_shape=[
            jax.ShapeDtypeStruct((N, D), jnp.float32),
            jax.ShapeDtypeStruct((N, D), jnp.float32),
        ],
    )(x, W, W_skip, b2)

    zero_block = jnp.zeros((RCH, D), jnp.float32)
    agg = _sc_spmm(support1, src, dst, ew, zero_block)

    out = pl.pallas_call(
        _combine_body,
        grid=(N // MMB,),
        in_specs=[
            pl.BlockSpec((NC, MMB, D), lambda i: (0, i, 0)),
            pl.BlockSpec((MMB, D), lambda i: (i, 0)),
        ],
        out_specs=pl.BlockSpec((MMB, D), lambda i: (i, 0)),
        out_shape=jax.ShapeDtypeStruct((N, D), jnp.float32),
    )(agg, support2b)
    return out
